# baseline (speedup 1.0000x reference)
import jax
import jax.numpy as jnp
from jax import lax
from jax.experimental import pallas as pl
from jax.experimental.pallas import tpu as pltpu
from jax.experimental.pallas import tpu_sc as plsc

_B = 8
_H = 32
_W = 32
_D = 256
_HW = _H * _W
_NCH = 2 * _D
_RPW = 32


def _sc_body(col_hbm, row_hbm, out_hbm, patt_v, rbuf_v, sem):
    cid = lax.axis_index("c")
    sid = lax.axis_index("s")
    wid = cid * 16 + sid

    pltpu.sync_copy(col_hbm, patt_v.at[:, pl.ds(0, _D)])
    pltpu.sync_copy(row_hbm.at[wid], rbuf_v)
    vregs = [rbuf_v[pl.ds(g * 16, 16)] for g in range(_D // 16)]
    for j in range(_RPW):
        for g in range(_D // 16):
            patt_v[j, pl.ds(_D + g * 16, 16)] = vregs[g]

    copies = [
        pltpu.make_async_copy(
            patt_v, out_hbm.at[b, pl.ds(wid * _RPW, _RPW), :], sem
        )
        for b in range(_B)
    ]
    for c in copies:
        c.start()
    for c in copies:
        c.wait()


def kernel(x, row_embed, col_embed):
    b = x.shape[0]
    h, w = x.shape[-2], x.shape[-1]
    d = col_embed.shape[-1]
    col = col_embed[:w]
    row = row_embed[:h]
    mesh = plsc.VectorSubcoreMesh(core_axis_name="c", subcore_axis_name="s")
    run = pl.kernel(
        _sc_body,
        mesh=mesh,
        out_type=jax.ShapeDtypeStruct((b, h * w, 2 * d), jnp.float32),
        scratch_types=[
            pltpu.VMEM((_RPW, _NCH), jnp.float32),
            pltpu.VMEM((_D,), jnp.float32),
            pltpu.SemaphoreType.DMA,
        ],
    )
    out = run(col, row)
    return out.reshape(b, h, w, 2 * d).transpose(0, 3, 1, 2)
